# baseline (speedup 1.0000x reference)
import functools

import jax
import jax.numpy as jnp
from jax.experimental import pallas as pl
from jax.experimental.pallas import tpu as pltpu

H = 32
VOCAB = 64
LANES = 128


def _tree_sum(xs):
    while len(xs) > 1:
        xs = [xs[i] + xs[i + 1] for i in range(0, len(xs), 2)]
    return xs[0]


def _mm_kernel(tok_ref, embT_ref, W1T_ref, b1_ref, W2T_ref, b2_ref,
               g_ref, be_ref, WrT_ref, br_ref, WoT_ref, bo_ref,
               out_ref, mt_ref, kb_ref, rd_ref, hid_ref,
               *, chunk, num_chunks):
    l = pl.program_id(1)
    n = chunk * LANES

    embT = embT_ref[...]
    z1 = jnp.maximum(
        jnp.dot(W1T_ref[...], embT, preferred_element_type=jnp.float32)
        + b1_ref[...], 0.0)
    ff = jnp.dot(W2T_ref[...], z1,
                 preferred_element_type=jnp.float32) + b2_ref[...]
    x = embT + ff
    mu = jnp.mean(x, axis=0, keepdims=True)
    var = jnp.mean((x - mu) * (x - mu), axis=0, keepdims=True)
    tableT = (x - mu) * jax.lax.rsqrt(var + 1e-5) * g_ref[...] + be_ref[...]
    rd_tab = 1.0 / (jnp.sum(tableT * tableT, axis=0, keepdims=True) + 1e-6)

    lhs = jnp.concatenate([
        jnp.broadcast_to(tableT[:, None, :], (H, 8, VOCAB)).reshape(8 * H, VOCAB),
        jnp.broadcast_to(rd_tab, (8, VOCAB)),
        tableT,
    ], axis=0)

    tok = tok_ref[0, 0]
    iota = jax.lax.broadcasted_iota(jnp.int32, (VOCAB, n), 0)
    onehot = jnp.where(iota == tok, 1.0, 0.0)
    res = jnp.dot(lhs, onehot, preferred_element_type=jnp.float32)
    kb_ref[...] = res[:8 * H].reshape(H, 8, n)
    rd_ref[...] = res[8 * H:8 * H + 8]
    hid_ref[...] = res[8 * H + 8:].reshape(H // 8, 8, n)

    @pl.when(l == 0)
    def _init():
        mt_ref[...] = jnp.zeros_like(mt_ref)

    def _predict(kb):
        return _tree_sum([mt_ref[j] * kb[j][None] for j in range(H)])

    def _slices(t):
        base = t * LANES
        kb = kb_ref[:, :, pl.ds(base, LANES)]
        kp = hid_ref[:, :, pl.ds(base, LANES)]
        rd = rd_ref[:, pl.ds(base, LANES)][None]
        return kb, kp, rd

    def minichunk(m, carry):
        t = 2 * m
        kb1, kp1, rd1 = _slices(t)
        kb2, kp2, rd2 = _slices(t + 1)
        w1 = _predict(kb1)
        w2 = _predict(kb2)
        g = jnp.sum(kp1.reshape(H, LANES) * kp2.reshape(H, LANES),
                    axis=0, keepdims=True)[None]
        d1 = kp1 - w1 * rd1
        d2 = kp2 - (w2 + d1 * g) * rd2
        for j in range(H):
            mt_ref[j] = mt_ref[j] + kb1[j][None] * d1 + kb2[j][None] * d2
        return carry

    def single(t):
        kb1, kp1, rd1 = _slices(t)
        w1 = _predict(kb1)
        d1 = kp1 - w1 * rd1
        for j in range(H):
            mt_ref[j] = mt_ref[j] + kb1[j][None] * d1

    jax.lax.fori_loop(0, chunk // 2 - 1, minichunk, 0, unroll=False)

    @pl.when(l != num_chunks - 1)
    def _full():
        minichunk(chunk // 2 - 1, 0)

    @pl.when(l == num_chunks - 1)
    def _readout():
        single(chunk - 2)
        kbq, _, _ = _slices(chunk - 1)
        ctx = _predict(kbq).reshape(H, LANES)
        y = jnp.dot(WrT_ref[...], ctx,
                    preferred_element_type=jnp.float32) + br_ref[...]
        out_ref[...] = jnp.dot(WoT_ref[...], y,
                               preferred_element_type=jnp.float32) + bo_ref[...]


@jax.jit
def kernel(seq, embed, W1, b1, W2, b2, gamma, beta, Wr, br, Wo, bo):
    B, L = seq.shape
    chunk = 128
    num_chunks = L // chunk
    nb = B // LANES
    n = chunk * LANES

    tok = seq.astype(jnp.int32).reshape(nb, LANES, num_chunks, chunk)
    tok = tok.transpose(0, 2, 3, 1).reshape(nb, num_chunks, 1, n)

    col = lambda v: v.reshape(-1, 1)
    wspec = lambda shape: pl.BlockSpec(shape, lambda i, j: (0, 0))

    out = pl.pallas_call(
        functools.partial(_mm_kernel, chunk=chunk, num_chunks=num_chunks),
        grid=(nb, num_chunks),
        in_specs=[
            pl.BlockSpec((1, 1, 1, n), lambda i, j: (i, j, 0, 0)),
            wspec((H, VOCAB)),
            wspec((2 * H, H)),
            wspec((2 * H, 1)),
            wspec((H, 2 * H)),
            wspec((H, 1)),
            wspec((H, 1)),
            wspec((H, 1)),
            wspec((H, H)),
            wspec((H, 1)),
            wspec((VOCAB, H)),
            wspec((VOCAB, 1)),
        ],
        out_specs=pl.BlockSpec((VOCAB, LANES), lambda i, j: (0, i)),
        out_shape=jax.ShapeDtypeStruct((VOCAB, B), jnp.float32),
        scratch_shapes=[
            pltpu.VMEM((H, H // 8, 8, LANES), jnp.float32),
            pltpu.VMEM((H, 8, n), jnp.float32),
            pltpu.VMEM((8, n), jnp.float32),
            pltpu.VMEM((H // 8, 8, n), jnp.float32),
        ],
        compiler_params=pltpu.CompilerParams(
            dimension_semantics=("parallel", "arbitrary"),
            vmem_limit_bytes=100 * 1024 * 1024,
        ),
    )(tok, embed.T, W1.T, col(b1), W2.T, col(b2), col(gamma), col(beta),
      Wr.T, col(br), Wo.T, col(bo))
    return out.T


# speedup vs baseline: 1.0467x; 1.0467x over previous
import functools

import jax
import jax.numpy as jnp
from jax.experimental import pallas as pl
from jax.experimental.pallas import tpu as pltpu

H = 32
VOCAB = 64
LANES = 128


def _tree_sum(xs):
    while len(xs) > 1:
        xs = [xs[i] + xs[i + 1] for i in range(0, len(xs), 2)]
    return xs[0]


def _mm_kernel(tok_ref, embT_ref, W1T_ref, b1_ref, W2T_ref, b2_ref,
               g_ref, be_ref, WrT_ref, br_ref, WoT_ref, bo_ref,
               out_ref, mt_ref, hid_ref, rd_ref,
               *, chunk, num_chunks):
    l = pl.program_id(1)
    n = chunk * LANES

    embT = embT_ref[...]
    z1 = jnp.maximum(
        jnp.dot(W1T_ref[...], embT, preferred_element_type=jnp.float32)
        + b1_ref[...], 0.0)
    ff = jnp.dot(W2T_ref[...], z1,
                 preferred_element_type=jnp.float32) + b2_ref[...]
    x = embT + ff
    mu = jnp.mean(x, axis=0, keepdims=True)
    var = jnp.mean((x - mu) * (x - mu), axis=0, keepdims=True)
    tableT = (x - mu) * jax.lax.rsqrt(var + 1e-5) * g_ref[...] + be_ref[...]
    rd_tab = 1.0 / (jnp.sum(tableT * tableT, axis=0, keepdims=True) + 1e-6)
    lhs = jnp.concatenate([tableT, rd_tab], axis=0)

    tok = tok_ref[0, 0]
    iota = jax.lax.broadcasted_iota(jnp.int32, (VOCAB, n), 0)
    onehot = jnp.where(iota == tok, 1.0, 0.0)
    res = jnp.dot(lhs, onehot, preferred_element_type=jnp.float32)
    hid_ref[...] = res[:H]
    rd_ref[...] = res[H:H + 1]

    @pl.when(l == 0)
    def _init():
        mt_ref[...] = jnp.zeros_like(mt_ref)

    def step(t, carry):
        base = t * LANES
        k = hid_ref[:, pl.ds(base, LANES)]
        rd = rd_ref[:, pl.ds(base, LANES)]
        MT = mt_ref[...]
        vp = _tree_sum([MT[j] * k[j:j + 1, :] for j in range(H)])
        delta = k - vp * rd
        mt_ref[...] = MT + k[:, None, :] * delta[None, :, :]
        return carry

    nsteps = jnp.where(l == num_chunks - 1, chunk - 1, chunk)
    jax.lax.fori_loop(0, nsteps, step, 0, unroll=False)

    @pl.when(l == num_chunks - 1)
    def _readout():
        q = hid_ref[:, pl.ds((chunk - 1) * LANES, LANES)]
        MT = mt_ref[...]
        ctx = _tree_sum([MT[j] * q[j:j + 1, :] for j in range(H)])
        y = jnp.dot(WrT_ref[...], ctx,
                    preferred_element_type=jnp.float32) + br_ref[...]
        out_ref[...] = jnp.dot(WoT_ref[...], y,
                               preferred_element_type=jnp.float32) + bo_ref[...]


@jax.jit
def kernel(seq, embed, W1, b1, W2, b2, gamma, beta, Wr, br, Wo, bo):
    B, L = seq.shape
    chunk = 128
    num_chunks = L // chunk
    nb = B // LANES
    n = chunk * LANES

    tok = seq.astype(jnp.int32).reshape(nb, LANES, num_chunks, chunk)
    tok = tok.transpose(0, 2, 3, 1).reshape(nb, num_chunks, 1, n)

    col = lambda v: v.reshape(-1, 1)
    wspec = lambda shape: pl.BlockSpec(shape, lambda i, j: (0, 0))

    out = pl.pallas_call(
        functools.partial(_mm_kernel, chunk=chunk, num_chunks=num_chunks),
        grid=(nb, num_chunks),
        in_specs=[
            pl.BlockSpec((1, 1, 1, n), lambda i, j: (i, j, 0, 0)),
            wspec((H, VOCAB)),
            wspec((2 * H, H)),
            wspec((2 * H, 1)),
            wspec((H, 2 * H)),
            wspec((H, 1)),
            wspec((H, 1)),
            wspec((H, 1)),
            wspec((H, H)),
            wspec((H, 1)),
            wspec((VOCAB, H)),
            wspec((VOCAB, 1)),
        ],
        out_specs=pl.BlockSpec((VOCAB, LANES), lambda i, j: (0, i)),
        out_shape=jax.ShapeDtypeStruct((VOCAB, B), jnp.float32),
        scratch_shapes=[
            pltpu.VMEM((H, H, LANES), jnp.float32),
            pltpu.VMEM((H, n), jnp.float32),
            pltpu.VMEM((1, n), jnp.float32),
        ],
        compiler_params=pltpu.CompilerParams(
            dimension_semantics=("parallel", "arbitrary"),
            vmem_limit_bytes=64 * 1024 * 1024,
        ),
    )(tok, embed.T, W1.T, col(b1), W2.T, col(b2), col(gamma), col(beta),
      Wr.T, col(br), Wo.T, col(bo))
    return out.T


# speedup vs baseline: 1.1339x; 1.0833x over previous
import functools

import jax
import jax.numpy as jnp
from jax.experimental import pallas as pl
from jax.experimental.pallas import tpu as pltpu

H = 32
VOCAB = 64
LANES = 128


def _tree_sum(xs):
    while len(xs) > 1:
        xs = [xs[i] + xs[i + 1] for i in range(0, len(xs), 2)]
    return xs[0]


def _mm_kernel(tok_ref, embT_ref, W1T_ref, b1_ref, W2T_ref, b2_ref,
               g_ref, be_ref, WrT_ref, br_ref, WoT_ref, bo_ref,
               out_ref, mt_ref, hid_ref, rd_ref,
               *, chunk, num_chunks):
    l = pl.program_id(1)
    n = chunk * LANES

    embT = embT_ref[...]
    z1 = jnp.maximum(
        jnp.dot(W1T_ref[...], embT, preferred_element_type=jnp.float32)
        + b1_ref[...], 0.0)
    ff = jnp.dot(W2T_ref[...], z1,
                 preferred_element_type=jnp.float32) + b2_ref[...]
    x = embT + ff
    mu = jnp.mean(x, axis=0, keepdims=True)
    var = jnp.mean((x - mu) * (x - mu), axis=0, keepdims=True)
    tableT = (x - mu) * jax.lax.rsqrt(var + 1e-5) * g_ref[...] + be_ref[...]
    rd_tab = 1.0 / (jnp.sum(tableT * tableT, axis=0, keepdims=True) + 1e-6)
    lhs = jnp.concatenate([tableT, rd_tab], axis=0)

    tok = tok_ref[0, 0]
    iota = jax.lax.broadcasted_iota(jnp.int32, (VOCAB, n), 0)
    onehot = jnp.where(iota == tok, 1.0, 0.0)
    res = jnp.dot(lhs, onehot, preferred_element_type=jnp.float32)
    hid_ref[...] = res[:H]
    rd_ref[...] = res[H:H + 1]

    @pl.when(l == 0)
    def _init():
        mt_ref[...] = jnp.zeros_like(mt_ref)

    def step(t, carry):
        base = t * LANES
        k = hid_ref[:, pl.ds(base, LANES)]
        rd = rd_ref[:, pl.ds(base, LANES)]
        MT = mt_ref[...]
        kjs = [k[j:j + 1, :] for j in range(H)]
        vp = _tree_sum([MT[j] * kjs[j] for j in range(H)])
        delta = k - vp * rd
        for j in range(H):
            mt_ref[j] = MT[j] + kjs[j] * delta
        return carry

    nsteps = jnp.where(l == num_chunks - 1, chunk - 1, chunk)
    jax.lax.fori_loop(0, nsteps, step, 0, unroll=False)

    @pl.when(l == num_chunks - 1)
    def _readout():
        q = hid_ref[:, pl.ds((chunk - 1) * LANES, LANES)]
        MT = mt_ref[...]
        ctx = _tree_sum([MT[j] * q[j:j + 1, :] for j in range(H)])
        y = jnp.dot(WrT_ref[...], ctx,
                    preferred_element_type=jnp.float32) + br_ref[...]
        out_ref[...] = jnp.dot(WoT_ref[...], y,
                               preferred_element_type=jnp.float32) + bo_ref[...]


@jax.jit
def kernel(seq, embed, W1, b1, W2, b2, gamma, beta, Wr, br, Wo, bo):
    B, L = seq.shape
    chunk = 128
    num_chunks = L // chunk
    nb = B // LANES
    n = chunk * LANES

    tok = seq.astype(jnp.int32).reshape(nb, LANES, num_chunks, chunk)
    tok = tok.transpose(0, 2, 3, 1).reshape(nb, num_chunks, 1, n)

    col = lambda v: v.reshape(-1, 1)
    wspec = lambda shape: pl.BlockSpec(shape, lambda i, j: (0, 0))

    out = pl.pallas_call(
        functools.partial(_mm_kernel, chunk=chunk, num_chunks=num_chunks),
        grid=(nb, num_chunks),
        in_specs=[
            pl.BlockSpec((1, 1, 1, n), lambda i, j: (i, j, 0, 0)),
            wspec((H, VOCAB)),
            wspec((2 * H, H)),
            wspec((2 * H, 1)),
            wspec((H, 2 * H)),
            wspec((H, 1)),
            wspec((H, 1)),
            wspec((H, 1)),
            wspec((H, H)),
            wspec((H, 1)),
            wspec((VOCAB, H)),
            wspec((VOCAB, 1)),
        ],
        out_specs=pl.BlockSpec((VOCAB, LANES), lambda i, j: (0, i)),
        out_shape=jax.ShapeDtypeStruct((VOCAB, B), jnp.float32),
        scratch_shapes=[
            pltpu.VMEM((H, H, LANES), jnp.float32),
            pltpu.VMEM((H, n), jnp.float32),
            pltpu.VMEM((1, n), jnp.float32),
        ],
        compiler_params=pltpu.CompilerParams(
            dimension_semantics=("parallel", "arbitrary"),
            vmem_limit_bytes=64 * 1024 * 1024,
        ),
    )(tok, embed.T, W1.T, col(b1), W2.T, col(b2), col(gamma), col(beta),
      Wr.T, col(br), Wo.T, col(bo))
    return out.T
